# baseline (speedup 1.0000x reference)
import functools

import jax
import jax.numpy as jnp
from jax import lax
from jax.experimental import pallas as pl
from jax.experimental.pallas import tpu as pltpu
from jax.experimental.pallas import tpu_sc as plsc

N = 10000
E = 320000
D = 128
C = 10
G = 128

NC = 2
NS = 16
NW = NC * NS

CH = 112
PHC = 24
W0 = 90
W1 = 90
TOTCH = NS * (W0 + W1)
E_PAD = TOTCH * CH
N_PAD = 10240
RPT = N_PAD // NS
NB_POOL = N_PAD // 128


def _segsum_body(hp_hbm, src_hbm, dst_hbm, out_hbm,
                 src_v, dst_v, gbuf0, gbuf1, mbuf0, mbuf1, agg_sh,
                 gs0, gs1, ms0, ms1):
    c = lax.axis_index("c")
    s = lax.axis_index("s")
    gbufs = (gbuf0, gbuf1)
    mbufs = (mbuf0, mbuf1)
    gsems = (gs0, gs1)
    msems = (ms0, ms1)

    zero16 = jnp.zeros((16,), jnp.float32)

    def _zrow(r, carry):
        for k in range(D // 16):
            mbuf0[r, pl.ds(k * 16, 16)] = zero16
        return carry

    lax.fori_loop(0, CH, _zrow, 0)
    row0 = s * RPT
    nfull = RPT // CH
    rem = RPT - nfull * CH
    for t in range(nfull):
        pltpu.sync_copy(mbuf0, agg_sh.at[pl.ds(row0 + t * CH, CH)])
    pltpu.sync_copy(mbuf0.at[pl.ds(0, rem)],
                    agg_sh.at[pl.ds(row0 + nfull * CH, rem)])
    plsc.subcore_barrier()

    cmask = jnp.full((16,), -65536, jnp.int32)

    def _iter(j, b):
        pltpu.make_async_copy(
            hp_hbm.at[pl.ds(0, CH)], gbufs[b], gsems[b]).wait()

        @pl.when(j + 1 < _iter.n)
        def _():
            pltpu.async_copy(
                hp_hbm.at[src_v.at[j + 1]], gbufs[1 - b], gsems[1 - b])

        @pl.when(j >= 2)
        def _():
            pltpu.make_async_copy(
                mbufs[b], agg_sh.at[pl.ds(row0, CH)], msems[b]).wait()

        def _crow(r4, carry):
            for dr in range(4):
                r = 4 * r4 + dr
                for k in range(D // 32):
                    w = gbufs[b][r, pl.ds(k * 16, 16)]
                    lo = plsc.bitcast(lax.shift_left(w, 16), jnp.float32)
                    hi = plsc.bitcast(lax.bitwise_and(w, cmask), jnp.float32)
                    mbufs[b][r, pl.ds(k * 16, 16)] = lo
                    mbufs[b][r, pl.ds(64 + k * 16, 16)] = hi
            return carry

        lax.fori_loop(0, CH // 4, _crow, 0)
        pltpu.async_copy(mbufs[b], agg_sh.at[dst_v.at[j]], msems[b], add=True)

    def _phase(n):
        _iter.n = n
        pltpu.async_copy(hp_hbm.at[src_v.at[0]], gbuf0, gs0)

        def _outer(g, carry):
            _iter(2 * g, 0)
            _iter(2 * g + 1, 1)
            return carry

        lax.fori_loop(0, n // 2, _outer, 0)
        pltpu.make_async_copy(mbuf0, agg_sh.at[pl.ds(row0, CH)], ms0).wait()
        pltpu.make_async_copy(mbuf1, agg_sh.at[pl.ds(row0, CH)], ms1).wait()

    def _run(start_chunk, W):
        done = 0
        while done < W:
            n = min(PHC, W - done)
            base = start_chunk + done
            pltpu.sync_copy(src_hbm.at[pl.ds(base, n)], src_v.at[pl.ds(0, n)])
            pltpu.sync_copy(dst_hbm.at[pl.ds(base, n)], dst_v.at[pl.ds(0, n)])
            _phase(n)
            done += n

    @pl.when(c == 0)
    def _():
        _run(s * W0, W0)

    @pl.when(c == 1)
    def _():
        _run(NS * W0 + s * W1, W1)

    plsc.subcore_barrier()

    rd = [None, None]
    bufs = (mbuf0, mbuf1)
    for t in range(nfull + 1):
        b = t % 2
        if rd[b] is not None:
            rd[b].wait()
        rr = row0 + t * CH
        nn = CH if t < nfull else rem
        pltpu.sync_copy(agg_sh.at[pl.ds(rr, nn)], bufs[b].at[pl.ds(0, nn)])
        rd[b] = pltpu.async_copy(
            bufs[b].at[pl.ds(0, nn)], out_hbm.at[c, pl.ds(rr, nn)], gsems[b])
    rd[0].wait()
    rd[1].wait()


_segsum = functools.partial(
    pl.kernel,
    out_type=jax.ShapeDtypeStruct((NC, N_PAD, D), jnp.float32),
    mesh=plsc.VectorSubcoreMesh(core_axis_name="c", subcore_axis_name="s"),
    compiler_params=pltpu.CompilerParams(
        use_tc_tiling_on_sc=False, needs_layout_passes=False),
    scratch_types=[
        pltpu.VMEM((PHC, CH), jnp.int32),
        pltpu.VMEM((PHC, CH), jnp.int32),
        pltpu.VMEM((CH, D // 2), jnp.int32),
        pltpu.VMEM((CH, D // 2), jnp.int32),
        pltpu.VMEM((CH, D), jnp.float32),
        pltpu.VMEM((CH, D), jnp.float32),
        pltpu.VMEM_SHARED((N_PAD, D), jnp.float32),
        pltpu.SemaphoreType.DMA,
        pltpu.SemaphoreType.DMA,
        pltpu.SemaphoreType.DMA,
        pltpu.SemaphoreType.DMA,
    ],
)(_segsum_body)


def _mm_body(parts_ref, w_ref, b_ref, o_ref, op_ref):
    acc = parts_ref[0] + parts_ref[1]
    y = jnp.dot(acc, w_ref[...], preferred_element_type=jnp.float32)
    y = jnp.maximum(y + b_ref[...], 0.0)
    o_ref[...] = y
    ua = lax.bitcast_convert_type(
        y[:, :64].astype(jnp.bfloat16), jnp.uint16).astype(jnp.uint32)
    ub = lax.bitcast_convert_type(
        y[:, 64:].astype(jnp.bfloat16), jnp.uint16).astype(jnp.uint32)
    op_ref[...] = lax.bitcast_convert_type(ua | (ub << 16), jnp.int32)


def _layer_mm(parts, W, b):
    blk = 1024
    return pl.pallas_call(
        _mm_body,
        grid=(N_PAD // blk,),
        in_specs=[
            pl.BlockSpec((NC, blk, D), lambda i: (0, i, 0)),
            pl.BlockSpec((D, D), lambda i: (0, 0)),
            pl.BlockSpec((1, D), lambda i: (0, 0)),
        ],
        out_specs=[
            pl.BlockSpec((blk, D), lambda i: (i, 0)),
            pl.BlockSpec((blk, D // 2), lambda i: (i, 0)),
        ],
        out_shape=[
            jax.ShapeDtypeStruct((N_PAD, D), jnp.float32),
            jax.ShapeDtypeStruct((N_PAD, D // 2), jnp.int32),
        ],
    )(parts, W, b.reshape(1, D))


def _pool_body(batch_ref, h_ref, wfc_ref, bfc_ref, o_ref, sums_ref, cnt_ref):
    i = pl.program_id(0)

    @pl.when(i == 0)
    def _():
        sums_ref[...] = jnp.zeros_like(sums_ref)
        cnt_ref[...] = jnp.zeros_like(cnt_ref)

    bvec = batch_ref[0]
    gid = lax.broadcasted_iota(jnp.int32, (G, 128), 0)
    mask = (gid == bvec).astype(jnp.float32)
    sums_ref[...] += jnp.dot(mask, h_ref[...], preferred_element_type=jnp.float32)
    cnt_ref[...] += jnp.sum(mask, axis=1, keepdims=True)

    @pl.when(i == NB_POOL - 1)
    def _():
        pooled = sums_ref[...] / jnp.maximum(cnt_ref[...], 1.0)
        logits = jnp.dot(pooled, wfc_ref[...], preferred_element_type=jnp.float32)
        logits = logits + bfc_ref[...]
        col = lax.broadcasted_iota(jnp.int32, (G, D), 1)
        valid = col < C
        neg = jnp.where(valid, logits, -jnp.inf)
        m = jnp.max(neg, axis=1, keepdims=True)
        ex = jnp.where(valid, jnp.exp(logits - m), 0.0)
        lse = jnp.log(jnp.sum(ex, axis=1, keepdims=True)) + m
        o_ref[...] = logits - lse


def _pool_head(batch3, h, wfc_p, bfc_p):
    return pl.pallas_call(
        _pool_body,
        grid=(NB_POOL,),
        in_specs=[
            pl.BlockSpec((1, 1, 128), lambda i: (i, 0, 0)),
            pl.BlockSpec((128, D), lambda i: (i, 0)),
            pl.BlockSpec((D, D), lambda i: (0, 0)),
            pl.BlockSpec((1, D), lambda i: (0, 0)),
        ],
        out_specs=pl.BlockSpec((G, D), lambda i: (0, 0)),
        out_shape=jax.ShapeDtypeStruct((G, D), jnp.float32),
        scratch_shapes=[
            pltpu.VMEM((G, D), jnp.float32),
            pltpu.VMEM((G, D), jnp.float32),
        ],
    )(batch3, h, wfc_p, bfc_p)


def kernel(x, edge_index, batch, W1_, b1, W2_, b2, W3_, b3, Wfc, bfc):
    src = edge_index[0]
    dst = edge_index[1]
    pad = E_PAD - E
    pad_dst = N + jnp.arange(pad, dtype=jnp.int32) % (N_PAD - N)
    src_flat = jnp.concatenate([src, jnp.zeros((pad,), jnp.int32)]).reshape(
        TOTCH, CH)
    dst_flat = jnp.concatenate([dst, pad_dst]).reshape(TOTCH, CH)
    h = jnp.pad(x, ((0, N_PAD - N), (0, 0)))
    batch3 = jnp.pad(batch, (0, N_PAD - N), constant_values=G).reshape(
        NB_POOL, 1, 128)
    wfc_p = jnp.pad(Wfc, ((0, 0), (0, D - C)))
    bfc_p = jnp.pad(bfc, (0, D - C)).reshape(1, D)

    ua = lax.bitcast_convert_type(
        h[:, :64].astype(jnp.bfloat16), jnp.uint16).astype(jnp.uint32)
    ub = lax.bitcast_convert_type(
        h[:, 64:].astype(jnp.bfloat16), jnp.uint16).astype(jnp.uint32)
    hp = lax.bitcast_convert_type(ua | (ub << 16), jnp.int32)

    for (W, b) in ((W1_, b1), (W2_, b2), (W3_, b3)):
        parts = _segsum(hp, src_flat, dst_flat)
        h, hp = _layer_mm(parts, W, b)
    out = _pool_head(batch3, h, wfc_p, bfc_p)
    return out[:, :C]


# speedup vs baseline: 1.0575x; 1.0575x over previous
import functools

import jax
import jax.numpy as jnp
from jax import lax
from jax.experimental import pallas as pl
from jax.experimental.pallas import tpu as pltpu
from jax.experimental.pallas import tpu_sc as plsc

N = 10000
E = 320000
D = 128
C = 10
G = 128

NC = 2
NS = 16
NW = NC * NS

CH = 112
PHC = 24
W0 = 90
W1 = 90
TOTCH = NS * (W0 + W1)
E_PAD = TOTCH * CH
N_PAD = 10240
RPT = N_PAD // NS
PBLK = 1024
NB_POOL = N_PAD // PBLK


def _segsum_body(hp_hbm, src_hbm, dst_hbm, out_hbm,
                 src_v, dst_v, gbuf0, gbuf1, mbuf0, mbuf1, agg_sh,
                 gs0, gs1, ms0, ms1):
    c = lax.axis_index("c")
    s = lax.axis_index("s")
    gbufs = (gbuf0, gbuf1)
    mbufs = (mbuf0, mbuf1)
    gsems = (gs0, gs1)
    msems = (ms0, ms1)

    zero16 = jnp.zeros((16,), jnp.float32)

    def _zrow(r, carry):
        for k in range(D // 16):
            mbuf0[r, pl.ds(k * 16, 16)] = zero16
        return carry

    lax.fori_loop(0, CH, _zrow, 0)
    row0 = s * RPT
    nfull = RPT // CH
    rem = RPT - nfull * CH
    for t in range(nfull):
        pltpu.sync_copy(mbuf0, agg_sh.at[pl.ds(row0 + t * CH, CH)])
    pltpu.sync_copy(mbuf0.at[pl.ds(0, rem)],
                    agg_sh.at[pl.ds(row0 + nfull * CH, rem)])
    plsc.subcore_barrier()

    cmask = jnp.full((16,), -65536, jnp.int32)

    def _iter(j, b):
        pltpu.make_async_copy(
            hp_hbm.at[pl.ds(0, CH)], gbufs[b], gsems[b]).wait()

        @pl.when(j + 1 < _iter.n)
        def _():
            pltpu.async_copy(
                hp_hbm.at[src_v.at[j + 1]], gbufs[1 - b], gsems[1 - b])

        @pl.when(j >= 2)
        def _():
            pltpu.make_async_copy(
                mbufs[b], agg_sh.at[pl.ds(row0, CH)], msems[b]).wait()

        def _crow(r, carry):
            for k in range(D // 32):
                w = gbufs[b][r, pl.ds(k * 16, 16)]
                lo = plsc.bitcast(lax.shift_left(w, 16), jnp.float32)
                hi = plsc.bitcast(lax.bitwise_and(w, cmask), jnp.float32)
                mbufs[b][r, pl.ds(k * 16, 16)] = lo
                mbufs[b][r, pl.ds(64 + k * 16, 16)] = hi
            return carry

        lax.fori_loop(0, CH, _crow, 0)
        pltpu.async_copy(mbufs[b], agg_sh.at[dst_v.at[j]], msems[b], add=True)

    def _phase(n):
        _iter.n = n
        pltpu.async_copy(hp_hbm.at[src_v.at[0]], gbuf0, gs0)

        def _outer(g, carry):
            _iter(2 * g, 0)
            _iter(2 * g + 1, 1)
            return carry

        lax.fori_loop(0, n // 2, _outer, 0)
        pltpu.make_async_copy(mbuf0, agg_sh.at[pl.ds(row0, CH)], ms0).wait()
        pltpu.make_async_copy(mbuf1, agg_sh.at[pl.ds(row0, CH)], ms1).wait()

    def _run(start_chunk, W):
        done = 0
        while done < W:
            n = min(PHC, W - done)
            base = start_chunk + done
            pltpu.sync_copy(src_hbm.at[pl.ds(base, n)], src_v.at[pl.ds(0, n)])
            pltpu.sync_copy(dst_hbm.at[pl.ds(base, n)], dst_v.at[pl.ds(0, n)])
            _phase(n)
            done += n

    @pl.when(c == 0)
    def _():
        _run(s * W0, W0)

    @pl.when(c == 1)
    def _():
        _run(NS * W0 + s * W1, W1)

    plsc.subcore_barrier()

    rd = [None, None]
    bufs = (mbuf0, mbuf1)
    for t in range(nfull + 1):
        b = t % 2
        if rd[b] is not None:
            rd[b].wait()
        rr = row0 + t * CH
        nn = CH if t < nfull else rem
        pltpu.sync_copy(agg_sh.at[pl.ds(rr, nn)], bufs[b].at[pl.ds(0, nn)])
        rd[b] = pltpu.async_copy(
            bufs[b].at[pl.ds(0, nn)], out_hbm.at[c, pl.ds(rr, nn)], gsems[b])
    rd[0].wait()
    rd[1].wait()


_segsum = functools.partial(
    pl.kernel,
    out_type=jax.ShapeDtypeStruct((NC, N_PAD, D), jnp.float32),
    mesh=plsc.VectorSubcoreMesh(core_axis_name="c", subcore_axis_name="s"),
    compiler_params=pltpu.CompilerParams(
        use_tc_tiling_on_sc=False, needs_layout_passes=False),
    scratch_types=[
        pltpu.VMEM((PHC, CH), jnp.int32),
        pltpu.VMEM((PHC, CH), jnp.int32),
        pltpu.VMEM((CH, D // 2), jnp.int32),
        pltpu.VMEM((CH, D // 2), jnp.int32),
        pltpu.VMEM((CH, D), jnp.float32),
        pltpu.VMEM((CH, D), jnp.float32),
        pltpu.VMEM_SHARED((N_PAD, D), jnp.float32),
        pltpu.SemaphoreType.DMA,
        pltpu.SemaphoreType.DMA,
        pltpu.SemaphoreType.DMA,
        pltpu.SemaphoreType.DMA,
    ],
)(_segsum_body)


def _mm_body(parts_ref, w_ref, b_ref, o_ref, op_ref):
    acc = parts_ref[0] + parts_ref[1]
    y = jnp.dot(acc, w_ref[...], preferred_element_type=jnp.float32)
    y = jnp.maximum(y + b_ref[...], 0.0)
    o_ref[...] = y
    ua = lax.bitcast_convert_type(
        y[:, :64].astype(jnp.bfloat16), jnp.uint16).astype(jnp.uint32)
    ub = lax.bitcast_convert_type(
        y[:, 64:].astype(jnp.bfloat16), jnp.uint16).astype(jnp.uint32)
    op_ref[...] = lax.bitcast_convert_type(ua | (ub << 16), jnp.int32)


def _layer_mm(parts, W, b):
    blk = 1024
    return pl.pallas_call(
        _mm_body,
        grid=(N_PAD // blk,),
        in_specs=[
            pl.BlockSpec((NC, blk, D), lambda i: (0, i, 0)),
            pl.BlockSpec((D, D), lambda i: (0, 0)),
            pl.BlockSpec((1, D), lambda i: (0, 0)),
        ],
        out_specs=[
            pl.BlockSpec((blk, D), lambda i: (i, 0)),
            pl.BlockSpec((blk, D // 2), lambda i: (i, 0)),
        ],
        out_shape=[
            jax.ShapeDtypeStruct((N_PAD, D), jnp.float32),
            jax.ShapeDtypeStruct((N_PAD, D // 2), jnp.int32),
        ],
    )(parts, W, b.reshape(1, D))


def _pool_body(batch_ref, h_ref, wfc_ref, bfc_ref, o_ref, sums_ref, cnt_ref):
    i = pl.program_id(0)

    @pl.when(i == 0)
    def _():
        sums_ref[...] = jnp.zeros_like(sums_ref)
        cnt_ref[...] = jnp.zeros_like(cnt_ref)

    bvec = batch_ref[0]
    gid = lax.broadcasted_iota(jnp.int32, (G, PBLK), 0)
    mask = (gid == bvec).astype(jnp.float32)
    sums_ref[...] += jnp.dot(mask, h_ref[...], preferred_element_type=jnp.float32)
    cnt_ref[...] += jnp.sum(mask, axis=1, keepdims=True)

    @pl.when(i == NB_POOL - 1)
    def _():
        pooled = sums_ref[...] / jnp.maximum(cnt_ref[...], 1.0)
        logits = jnp.dot(pooled, wfc_ref[...], preferred_element_type=jnp.float32)
        logits = logits + bfc_ref[...]
        col = lax.broadcasted_iota(jnp.int32, (G, D), 1)
        valid = col < C
        neg = jnp.where(valid, logits, -jnp.inf)
        m = jnp.max(neg, axis=1, keepdims=True)
        ex = jnp.where(valid, jnp.exp(logits - m), 0.0)
        lse = jnp.log(jnp.sum(ex, axis=1, keepdims=True)) + m
        o_ref[...] = logits - lse


def _pool_head(batch3, h, wfc_p, bfc_p):
    return pl.pallas_call(
        _pool_body,
        grid=(NB_POOL,),
        in_specs=[
            pl.BlockSpec((1, 1, PBLK), lambda i: (i, 0, 0)),
            pl.BlockSpec((PBLK, D), lambda i: (i, 0)),
            pl.BlockSpec((D, D), lambda i: (0, 0)),
            pl.BlockSpec((1, D), lambda i: (0, 0)),
        ],
        out_specs=pl.BlockSpec((G, D), lambda i: (0, 0)),
        out_shape=jax.ShapeDtypeStruct((G, D), jnp.float32),
        scratch_shapes=[
            pltpu.VMEM((G, D), jnp.float32),
            pltpu.VMEM((G, D), jnp.float32),
        ],
    )(batch3, h, wfc_p, bfc_p)


def kernel(x, edge_index, batch, W1_, b1, W2_, b2, W3_, b3, Wfc, bfc):
    src = edge_index[0]
    dst = edge_index[1]
    pad = E_PAD - E
    pad_dst = N + jnp.arange(pad, dtype=jnp.int32) % (N_PAD - N)
    src_flat = jnp.concatenate([src, jnp.zeros((pad,), jnp.int32)]).reshape(
        TOTCH, CH)
    dst_flat = jnp.concatenate([dst, pad_dst]).reshape(TOTCH, CH)
    h = jnp.pad(x, ((0, N_PAD - N), (0, 0)))
    batch3 = jnp.pad(batch, (0, N_PAD - N), constant_values=G).reshape(
        NB_POOL, 1, PBLK)
    wfc_p = jnp.pad(Wfc, ((0, 0), (0, D - C)))
    bfc_p = jnp.pad(bfc, (0, D - C)).reshape(1, D)

    ua = lax.bitcast_convert_type(
        h[:, :64].astype(jnp.bfloat16), jnp.uint16).astype(jnp.uint32)
    ub = lax.bitcast_convert_type(
        h[:, 64:].astype(jnp.bfloat16), jnp.uint16).astype(jnp.uint32)
    hp = lax.bitcast_convert_type(ua | (ub << 16), jnp.int32)

    for (W, b) in ((W1_, b1), (W2_, b2), (W3_, b3)):
        parts = _segsum(hp, src_flat, dst_flat)
        h, hp = _layer_mm(parts, W, b)
    out = _pool_head(batch3, h, wfc_p, bfc_p)
    return out[:, :C]
